# initial kernel scaffold (unmeasured)
import jax
import jax.numpy as jnp
from jax import lax
from jax.experimental import pallas as pl
from jax.experimental.pallas import tpu as pltpu

N_DEV = 8


def kernel(x, w_mat):
    m_per, k = x.shape
    n = w_mat.shape[1]
    n_per = n // N_DEV
    out_dtype = jnp.bfloat16

    def body(x_ref, w_ref, out_ref, xb_ref, y_ref, send_sems, recv_sems):
        my = lax.axis_index("i")

        xb_ref[...] = x_ref[...].astype(jnp.bfloat16)

        for d in range(N_DEV):
            wb = w_ref[:, d * n_per:(d + 1) * n_per].astype(jnp.bfloat16)
            acc = jnp.dot(xb_ref[...], wb, preferred_element_type=jnp.float32)
            y_ref[d] = jnp.maximum(acc, 0.0).astype(out_dtype)

        out_ref[pl.ds(my * m_per, m_per), :] = y_ref[my]

        sends = []
        for off in range(1, N_DEV):
            dst = lax.rem(my + off, N_DEV)
            rdma = pltpu.make_async_remote_copy(
                src_ref=y_ref.at[dst],
                dst_ref=out_ref.at[pl.ds(my * m_per, m_per), :],
                send_sem=send_sems.at[off],
                recv_sem=recv_sems.at[off],
                device_id=(dst,),
                device_id_type=pl.DeviceIdType.MESH,
            )
            rdma.start()
            sends.append(rdma)

        for off in range(1, N_DEV):
            src = lax.rem(my - off + N_DEV, N_DEV)
            recv = pltpu.make_async_remote_copy(
                src_ref=y_ref.at[src],
                dst_ref=out_ref.at[pl.ds(src * m_per, m_per), :],
                send_sem=send_sems.at[off],
                recv_sem=recv_sems.at[off],
                device_id=(src,),
                device_id_type=pl.DeviceIdType.MESH,
            )
            recv.wait_recv()

        for rdma in sends:
            rdma.wait_send()

    return pl.pallas_call(
        body,
        out_shape=jax.ShapeDtypeStruct((N_DEV * m_per, n_per), out_dtype),
        in_specs=[
            pl.BlockSpec(memory_space=pltpu.VMEM),
            pl.BlockSpec(memory_space=pltpu.VMEM),
        ],
        out_specs=pl.BlockSpec(memory_space=pltpu.VMEM),
        scratch_shapes=[
            pltpu.VMEM((m_per, k), jnp.bfloat16),
            pltpu.VMEM((N_DEV, m_per, n_per), out_dtype),
            pltpu.SemaphoreType.DMA((N_DEV,)),
            pltpu.SemaphoreType.DMA((N_DEV,)),
        ],
    )(x, w_mat)


# baseline (device time: 49653 ns/iter reference)
import jax
import jax.numpy as jnp
from jax import lax
from jax.experimental import pallas as pl
from jax.experimental.pallas import tpu as pltpu

N_DEV = 8


def kernel(x, w_mat):
    m_per, k = x.shape
    n = w_mat.shape[1]
    n_per = n // N_DEV
    out_dtype = jnp.bfloat16

    def body(x_ref, w_ref, out_ref, xb_ref, y_ref, send_sems, recv_sems):
        my = lax.axis_index("i")

        xb_ref[...] = x_ref[...].astype(jnp.bfloat16)

        for d in range(N_DEV):
            wb = w_ref[:, d * n_per:(d + 1) * n_per].astype(jnp.bfloat16)
            acc = jnp.dot(xb_ref[...], wb, preferred_element_type=jnp.float32)
            y_ref[d] = jnp.maximum(acc, 0.0).astype(out_dtype)

        out_ref[pl.ds(my * m_per, m_per), :] = y_ref[my]

        sends = []
        for off in range(1, N_DEV):
            dst = lax.rem(my + off, N_DEV)
            rdma = pltpu.make_async_remote_copy(
                src_ref=y_ref.at[dst],
                dst_ref=out_ref.at[pl.ds(my * m_per, m_per), :],
                send_sem=send_sems.at[off],
                recv_sem=recv_sems.at[off],
                device_id=(dst,),
                device_id_type=pl.DeviceIdType.MESH,
            )
            rdma.start()
            sends.append(rdma)

        for off in range(1, N_DEV):
            src = lax.rem(my - off + N_DEV, N_DEV)
            recv = pltpu.make_async_remote_copy(
                src_ref=y_ref.at[src],
                dst_ref=out_ref.at[pl.ds(src * m_per, m_per), :],
                send_sem=send_sems.at[off],
                recv_sem=recv_sems.at[off],
                device_id=(src,),
                device_id_type=pl.DeviceIdType.MESH,
            )
            recv.wait_recv()

        for rdma in sends:
            rdma.wait_send()

    return pl.pallas_call(
        body,
        out_shape=jax.ShapeDtypeStruct((N_DEV * m_per, n_per), out_dtype),
        in_specs=[
            pl.BlockSpec(memory_space=pltpu.VMEM),
            pl.BlockSpec(memory_space=pltpu.VMEM),
        ],
        out_specs=pl.BlockSpec(memory_space=pltpu.VMEM),
        scratch_shapes=[
            pltpu.VMEM((m_per, k), jnp.bfloat16),
            pltpu.VMEM((N_DEV, m_per, n_per), out_dtype),
            pltpu.SemaphoreType.DMA((N_DEV,)),
            pltpu.SemaphoreType.DMA((N_DEV,)),
        ],
        compiler_params=pltpu.CompilerParams(
            vmem_limit_bytes=100 * 1024 * 1024,
        ),
    )(x, w_mat)


# device time: 39076 ns/iter; 1.2707x vs baseline; 1.2707x over previous
import os

import jax
import jax.numpy as jnp
from jax import lax
from jax.experimental import pallas as pl
from jax.experimental.pallas import tpu as pltpu

N_DEV = 8
_NO_COMM = os.environ.get("KERNEL_NO_COMM", "0") == "1"
_NO_COMPUTE = os.environ.get("KERNEL_NO_COMPUTE", "0") == "1"


def kernel(x, w_mat):
    m_per, k = x.shape
    n = w_mat.shape[1]
    n_per = n // N_DEV
    out_dtype = jnp.bfloat16

    order = list(range(1, N_DEV)) + [0]

    def body(x_ref, w_ref, out_ref, wv_ref, y_ref, dma_sems, send_sems,
             recv_sems):
        my = lax.axis_index("i")

        def w_dma(g, slot):
            col = lax.rem(my + g, N_DEV) * n_per
            return pltpu.make_async_copy(
                w_ref.at[:, pl.ds(col, n_per)], wv_ref.at[slot],
                dma_sems.at[slot],
            )

        w_dma(order[0], 0).start()

        sends = []
        for i, g in enumerate(order):
            slot = i % 2
            if i + 1 < N_DEV:
                w_dma(order[i + 1], (i + 1) % 2).start()
            w_dma(g, slot).wait()

            if _NO_COMPUTE:
                continue
            acc = jnp.dot(x_ref[...], wv_ref[slot],
                          preferred_element_type=jnp.float32)
            yc = jnp.maximum(acc, 0.0).astype(out_dtype)

            if g == 0:
                out_ref[pl.ds(my * m_per, m_per), :] = yc
            else:
                y_ref[g] = yc
                if _NO_COMM:
                    continue
                dst = lax.rem(my + g, N_DEV)
                rdma = pltpu.make_async_remote_copy(
                    src_ref=y_ref.at[g],
                    dst_ref=out_ref.at[pl.ds(my * m_per, m_per), :],
                    send_sem=send_sems.at[g],
                    recv_sem=recv_sems.at[g],
                    device_id=(dst,),
                    device_id_type=pl.DeviceIdType.MESH,
                )
                rdma.start()
                sends.append(rdma)

        if _NO_COMM or _NO_COMPUTE:
            return

        for g in range(1, N_DEV):
            src = lax.rem(my - g + N_DEV, N_DEV)
            recv = pltpu.make_async_remote_copy(
                src_ref=y_ref.at[g],
                dst_ref=out_ref.at[pl.ds(src * m_per, m_per), :],
                send_sem=send_sems.at[g],
                recv_sem=recv_sems.at[g],
                device_id=(src,),
                device_id_type=pl.DeviceIdType.MESH,
            )
            recv.wait_recv()

        for rdma in sends:
            rdma.wait_send()

    return pl.pallas_call(
        body,
        out_shape=jax.ShapeDtypeStruct((N_DEV * m_per, n_per), out_dtype),
        in_specs=[
            pl.BlockSpec(memory_space=pltpu.VMEM),
            pl.BlockSpec(memory_space=pltpu.MemorySpace.HBM),
        ],
        out_specs=pl.BlockSpec(memory_space=pltpu.VMEM),
        scratch_shapes=[
            pltpu.VMEM((2, k, n_per), jnp.float32),
            pltpu.VMEM((N_DEV, m_per, n_per), out_dtype),
            pltpu.SemaphoreType.DMA((2,)),
            pltpu.SemaphoreType.DMA((N_DEV,)),
            pltpu.SemaphoreType.DMA((N_DEV,)),
        ],
        compiler_params=pltpu.CompilerParams(
            vmem_limit_bytes=100 * 1024 * 1024,
        ),
    )(x, w_mat)
